# Initial kernel scaffold; baseline (speedup 1.0000x reference)
import functools

import jax
import jax.numpy as jnp
from jax import lax
from jax.experimental import pallas as pl
from jax.experimental.pallas import tpu as pltpu
from jax.experimental.pallas import tpu_sc as plsc

H = 4
D = 32
LANES = 16
CHUNK = 128


def _make_sc_lookup(n_total, n_vocab):
    info = plsc.get_sparse_core_info()
    nc, ns = info.num_cores, info.num_subcores
    nw = nc * ns
    assert n_total % (nw * CHUNK) == 0
    n_per_w = n_total // nw
    n_chunks = n_per_w // CHUNK

    mesh = plsc.VectorSubcoreMesh(core_axis_name="c", subcore_axis_name="s")

    @functools.partial(
        pl.kernel,
        mesh=mesh,
        out_type=jax.ShapeDtypeStruct((n_total, D), jnp.float32),
        scratch_types=[
            pltpu.VMEM((H, CHUNK), jnp.int32),
            pltpu.VMEM((H, CHUNK, D), jnp.float32),
            pltpu.VMEM((CHUNK, D), jnp.float32),
            pltpu.SemaphoreType.DMA,
        ],
    )
    def k(x_hbm, tab_hbm, out_hbm, idx4, buf, acc, sem):
        wid = lax.axis_index("s") * nc + lax.axis_index("c")
        base = wid * n_per_w

        def chunk_body(g, _):
            row0 = base + g * CHUNK
            pltpu.sync_copy(x_hbm.at[pl.ds(row0, CHUNK)], idx4.at[0])

            def build_idx(j, _):
                v = idx4[0, pl.ds(j * LANES, LANES)]
                idx4[1, pl.ds(j * LANES, LANES)] = v + n_vocab
                idx4[2, pl.ds(j * LANES, LANES)] = v + 2 * n_vocab
                idx4[3, pl.ds(j * LANES, LANES)] = v + 3 * n_vocab
                return 0

            lax.fori_loop(0, CHUNK // LANES, build_idx, 0, unroll=True)

            cps = [
                pltpu.async_copy(tab_hbm.at[idx4.at[h]], buf.at[h], sem)
                for h in range(H)
            ]
            for cp in cps:
                cp.wait()

            def sum_body(r, _):
                for d0 in range(0, D, LANES):
                    s = buf[0, r, pl.ds(d0, LANES)]
                    s = s + buf[1, r, pl.ds(d0, LANES)]
                    s = s + buf[2, r, pl.ds(d0, LANES)]
                    s = s + buf[3, r, pl.ds(d0, LANES)]
                    acc[r, pl.ds(d0, LANES)] = s
                return 0

            lax.fori_loop(0, CHUNK, sum_body, 0, unroll=4)

            pltpu.sync_copy(acc, out_hbm.at[pl.ds(row0, CHUNK)])
            return 0

        lax.fori_loop(0, n_chunks, chunk_body, 0)

    return k


def kernel(x, tables):
    b, s = x.shape
    h, v, d = tables.shape
    n = b * s
    xf = x.reshape(n)
    tf = tables.reshape(h * v, d)
    out = _make_sc_lookup(n, v)(xf, tf)
    return out.reshape(b, s, d)


# speedup vs baseline: 13.1730x; 13.1730x over previous
import functools

import jax
import jax.numpy as jnp
from jax import lax
from jax.experimental import pallas as pl
from jax.experimental.pallas import tpu as pltpu
from jax.experimental.pallas import tpu_sc as plsc

H = 4
D = 32
LANES = 16
CHUNK = 128
NA = 4
K = 2


def _make_sc_lookup(n_total):
    info = plsc.get_sparse_core_info()
    nc, ns = info.num_cores, info.num_subcores
    nw = nc * ns
    assert n_total % (nw * CHUNK) == 0
    n_per_w = n_total // nw
    n_chunks = n_per_w // CHUNK
    assert n_chunks % NA == 0 and n_chunks >= 2 * NA

    mesh = plsc.VectorSubcoreMesh(core_axis_name="c", subcore_axis_name="s")

    @functools.partial(
        pl.kernel,
        mesh=mesh,
        compiler_params=pltpu.CompilerParams(use_tc_tiling_on_sc=False),
        out_type=jax.ShapeDtypeStruct((n_total, D), jnp.float32),
        scratch_types=[
            pltpu.VMEM((n_chunks, CHUNK), jnp.int32),
            pltpu.VMEM((NA, CHUNK, D), jnp.float32),
            [pltpu.SemaphoreType.DMA] * NA,
            [pltpu.SemaphoreType.DMA] * NA,
        ],
    )
    def k(x_hbm, t0, t1, t2, t3, out_hbm, idx_all, acc, gsems, ssems):
        tabs = (t0, t1, t2, t3)
        wid = lax.axis_index("s") * nc + lax.axis_index("c")
        base = wid * n_per_w
        chunk0 = wid * n_chunks

        pltpu.sync_copy(x_hbm.at[pl.ds(chunk0, n_chunks)], idx_all)

        zeros = jnp.zeros((LANES,), jnp.float32)

        def zero_acc(a):
            def zbody(r, _):
                acc[a, r, pl.ds(0, LANES)] = zeros
                acc[a, r, pl.ds(LANES, LANES)] = zeros
                return 0
            lax.fori_loop(0, CHUNK, zbody, 0, unroll=8)

        def issue_gathers(g, a):
            idx = idx_all.at[g]
            for h in range(H):
                pltpu.async_copy(tabs[h].at[idx], acc.at[a], gsems[a], add=True)

        def drain_and_store(j, a):
            cp = pltpu.make_async_copy(tabs[0].at[idx_all.at[j]], acc.at[a], gsems[a])
            for _ in range(H):
                cp.wait()
            pltpu.async_copy(acc.at[a], out_hbm.at[pl.ds(base + j * CHUNK, CHUNK)],
                             ssems[a])

        def wait_store(a):
            pltpu.make_async_copy(
                acc.at[a], out_hbm.at[pl.ds(base, CHUNK)], ssems[a]).wait()

        for b in range(NA):
            zero_acc(b)
            issue_gathers(b, b)
            if b >= K:
                drain_and_store(b - K, b - K)

        def outer(g0, _):
            for b in range(NA):
                g = g0 * NA + b
                wait_store(b)
                zero_acc(b)
                issue_gathers(g, b)
                drain_and_store(g - K, (b - K) % NA)
            return 0

        lax.fori_loop(1, n_chunks // NA, outer, 0)

        for b in range(K):
            j = n_chunks - K + b
            drain_and_store(j, j % NA)
        for b in range(NA):
            wait_store((n_chunks - NA + b) % NA)

    return k


def kernel(x, tables):
    b, s = x.shape
    h, v, d = tables.shape
    n = b * s
    x2 = x.reshape(n // CHUNK, CHUNK)
    out = _make_sc_lookup(n)(x2, tables[0], tables[1], tables[2], tables[3])
    return out.reshape(b, s, d)


# speedup vs baseline: 17.5950x; 1.3357x over previous
import functools

import jax
import jax.numpy as jnp
from jax import lax
from jax.experimental import pallas as pl
from jax.experimental.pallas import tpu as pltpu
from jax.experimental.pallas import tpu_sc as plsc

H = 4
D = 32
LANES = 16
CHUNK = 128
NA = 4
K = 2


def _make_sc_lookup(n_total, n_vocab):
    info = plsc.get_sparse_core_info()
    nc, ns = info.num_cores, info.num_subcores
    nw = nc * ns
    assert n_total % (nw * CHUNK) == 0
    n_per_w = n_total // nw
    n_chunks = n_per_w // CHUNK
    assert n_chunks % NA == 0 and n_chunks >= 2 * NA

    mesh = plsc.VectorSubcoreMesh(core_axis_name="c", subcore_axis_name="s")

    @functools.partial(
        pl.kernel,
        mesh=mesh,
        compiler_params=pltpu.CompilerParams(use_tc_tiling_on_sc=False),
        out_type=jax.ShapeDtypeStruct((n_total, D), jnp.float32),
        scratch_types=[
            pltpu.VMEM((n_chunks, CHUNK), jnp.int32),
            pltpu.VMEM((NA, H - 1, CHUNK), jnp.int32),
            pltpu.VMEM((NA, CHUNK, D), jnp.float32),
            [pltpu.SemaphoreType.DMA] * NA,
            [pltpu.SemaphoreType.DMA] * NA,
        ],
    )
    def k(x_hbm, tab_hbm, out_hbm, idx_all, idx3, acc, gsems, ssems):
        wid = lax.axis_index("s") * nc + lax.axis_index("c")
        base = wid * n_per_w
        chunk0 = wid * n_chunks

        pltpu.sync_copy(x_hbm.at[pl.ds(chunk0, n_chunks)], idx_all)

        zeros = jnp.zeros((LANES,), jnp.float32)

        def prep(g, a):
            def ibody(j, _):
                v = idx_all[g, pl.ds(j * LANES, LANES)]
                idx3[a, 0, pl.ds(j * LANES, LANES)] = v + n_vocab
                idx3[a, 1, pl.ds(j * LANES, LANES)] = v + 2 * n_vocab
                idx3[a, 2, pl.ds(j * LANES, LANES)] = v + 3 * n_vocab
                return 0
            lax.fori_loop(0, CHUNK // LANES, ibody, 0, unroll=True)

            def zbody(r, _):
                acc[a, r, pl.ds(0, LANES)] = zeros
                acc[a, r, pl.ds(LANES, LANES)] = zeros
                return 0
            lax.fori_loop(0, CHUNK, zbody, 0, unroll=8)

        def issue_gathers(g, a):
            pltpu.async_copy(tab_hbm.at[idx_all.at[g]], acc.at[a], gsems[a],
                             add=True)
            for h in range(H - 1):
                pltpu.async_copy(tab_hbm.at[idx3.at[a, h]], acc.at[a], gsems[a],
                                 add=True)

        def drain_and_store(j, a):
            cp = pltpu.make_async_copy(tab_hbm.at[idx_all.at[j]], acc.at[a],
                                       gsems[a])
            for _ in range(H):
                cp.wait()
            pltpu.async_copy(acc.at[a], out_hbm.at[pl.ds(base + j * CHUNK, CHUNK)],
                             ssems[a])

        def wait_store(a):
            pltpu.make_async_copy(
                acc.at[a], out_hbm.at[pl.ds(base, CHUNK)], ssems[a]).wait()

        for b in range(NA):
            prep(b, b)
            issue_gathers(b, b)
            if b >= K:
                drain_and_store(b - K, b - K)

        def outer(g0, _):
            for b in range(NA):
                g = g0 * NA + b
                wait_store(b)
                prep(g, b)
                issue_gathers(g, b)
                drain_and_store(g - K, (b - K) % NA)
            return 0

        lax.fori_loop(1, n_chunks // NA, outer, 0)

        for b in range(K):
            j = n_chunks - K + b
            drain_and_store(j, j % NA)
        for b in range(NA):
            wait_store((n_chunks - NA + b) % NA)

    return k


def kernel(x, tables):
    b, s = x.shape
    h, v, d = tables.shape
    n = b * s
    x2 = x.reshape(n // CHUNK, CHUNK)
    tf = tables.reshape(h * v, d)
    out = _make_sc_lookup(n, v)(x2, tf)
    return out.reshape(b, s, d)


# speedup vs baseline: 45.8021x; 2.6031x over previous
import functools

import jax
import jax.numpy as jnp
from jax import lax
from jax.experimental import pallas as pl
from jax.experimental.pallas import tpu as pltpu
from jax.experimental.pallas import tpu_sc as plsc

H = 4
D = 32
LANES = 16
W = H * D
CHUNK = 128
NA = 4
PF = 3


def _make_sc_lookup(n_b, n_s):
    info = plsc.get_sparse_core_info()
    nc, ns = info.num_cores, info.num_subcores
    nw = nc * ns
    n_total = n_b * n_s
    cps = n_b // CHUNK
    assert n_total % (nw * CHUNK) == 0
    n_chunks = n_total // (nw * CHUNK)
    assert n_chunks % NA == 0 and n_chunks >= 2 * NA

    mesh = plsc.VectorSubcoreMesh(core_axis_name="c", subcore_axis_name="s")

    @functools.partial(
        pl.kernel,
        mesh=mesh,
        compiler_params=pltpu.CompilerParams(use_tc_tiling_on_sc=True,
                                             needs_layout_passes=False),
        out_type=jax.ShapeDtypeStruct((n_s, D, n_b), jnp.float32),
        scratch_types=[
            pltpu.VMEM((n_chunks, CHUNK), jnp.int32),
            pltpu.VMEM((NA, CHUNK, W), jnp.float32),
            pltpu.VMEM((NA, D, CHUNK), jnp.float32),
            [pltpu.SemaphoreType.DMA] * NA,
            [pltpu.SemaphoreType.DMA] * NA,
        ],
    )
    def k(x_hbm, tab_hbm, out_hbm, idx_all, buf, acc, gsems, ssems):
        wid = lax.axis_index("s") * nc + lax.axis_index("c")
        chunk0 = wid * n_chunks

        pltpu.sync_copy(x_hbm.at[pl.ds(chunk0, n_chunks)], idx_all)

        rows_lo = lax.iota(jnp.int32, LANES)
        rows_hi = rows_lo + LANES

        def issue_gather(g, a):
            pltpu.async_copy(tab_hbm.at[idx_all.at[g]], buf.at[a], gsems[a])

        def wait_gather(j, a):
            pltpu.make_async_copy(tab_hbm.at[idx_all.at[j]], buf.at[a],
                                  gsems[a]).wait()

        def sum_transpose(a):
            acc2 = acc.at[a]

            def body(r):
                col = jnp.zeros((LANES,), jnp.int32) + r
                lo = (buf[a, r, pl.ds(0, LANES)]
                      + buf[a, r, pl.ds(D, LANES)]
                      + buf[a, r, pl.ds(2 * D, LANES)]
                      + buf[a, r, pl.ds(3 * D, LANES)])
                hi = (buf[a, r, pl.ds(LANES, LANES)]
                      + buf[a, r, pl.ds(D + LANES, LANES)]
                      + buf[a, r, pl.ds(2 * D + LANES, LANES)]
                      + buf[a, r, pl.ds(3 * D + LANES, LANES)])
                plsc.store_scatter(acc2, [rows_lo, col], lo)
                plsc.store_scatter(acc2, [rows_hi, col], hi)

            plsc.parallel_loop(0, CHUNK, 1, unroll=16)(body)

        def issue_store(g, a):
            gc = chunk0 + g
            s = gc // cps
            b0 = gc % cps
            pltpu.async_copy(acc.at[a],
                             out_hbm.at[s, :, pl.ds(b0 * CHUNK, CHUNK)],
                             ssems[a])

        def wait_store(a):
            pltpu.make_async_copy(
                acc.at[a], out_hbm.at[0, :, pl.ds(0, CHUNK)], ssems[a]).wait()

        for p in range(PF):
            issue_gather(p, p)
        for b in range(NA):
            wait_gather(b, b)
            issue_gather(b + PF, (b + PF) % NA)
            sum_transpose(b)
            issue_store(b, b)

        def outer(g0, _):
            for b in range(NA):
                g = g0 * NA + b
                wait_gather(g, b)

                @pl.when(g + PF < n_chunks)
                def _():
                    issue_gather(g + PF, (b + PF) % NA)

                wait_store(b)
                sum_transpose(b)
                issue_store(g, b)
            return 0

        lax.fori_loop(1, n_chunks // NA, outer, 0)

        for b in range(NA):
            wait_store((n_chunks - NA + b) % NA)

    return k


def kernel(x, tables):
    b, s = x.shape
    h, v, d = tables.shape
    n = b * s
    x2 = x.T.reshape(n // CHUNK, CHUNK)
    ts = tables.transpose(1, 0, 2).reshape(v, h * d)
    out = _make_sc_lookup(b, s)(x2, ts)
    return out.transpose(2, 0, 1)
